# baseline (speedup 1.0000x reference)
import functools
import jax
import jax.numpy as jnp
from jax import lax
from jax.experimental import pallas as pl
from jax.experimental.pallas import tpu as pltpu
from jax.experimental.pallas import tpu_sc as plsc

N_NODES = 10000
N_EDGES = 320000
D = 128
K = 3
OUT = 10
NUM_GRAPHS = 64
EPS = 1e-15

NC = 2
NS = 16
NW = NC * NS
NPT = 320
NPAD = NW * NPT
SPARE = NPT
AGG_ROWS = NPT + 8
SCAN = 2000
FLUSH = 2048
BCAP = 4096 + 16
SUP = 512
CH = 128
RCAP = N_EDGES + 4096 + 16
NPRM = 18

_MESH = plsc.VectorSubcoreMesh(core_axis_name="c", subcore_axis_name="s")
_SC_PARAMS = pltpu.CompilerParams(needs_layout_passes=False)


def _iota16():
    return lax.iota(jnp.int32, 16)


_GDN = lax.GatherDimensionNumbers(
    offset_dims=(), collapsed_slice_dims=(0,), start_index_map=(0,))


def _lane(v, j):
    idx = jnp.full((16, 1), j, jnp.int32)
    return lax.gather(v, idx, _GDN, (1,),
                      mode=lax.GatherScatterMode.PROMISE_IN_BOUNDS)


def _wid():
    return lax.axis_index("s") * NC + lax.axis_index("c")


def _al8(x):
    return pl.multiple_of(x, 8)


def _tanh16(x):
    e = jnp.exp(jnp.clip(x + x, -40.0, 40.0))
    return (e - 1.0) / (e + 1.0)


def _compact_body(src_hbm, dst_hbm, ea0_hbm, ea1_hbm,
                  srcL, dstL, eaL0, eaL1, cnts2,
                  sbuf2, dbuf2, abuf0, abuf1,
                  bsrc, bdl, bea0, bea1, cbuf, sem):
    wid = _wid()
    lo = wid * NPT
    it = _iota16()
    dummy_i = jnp.zeros((16,), jnp.int32)
    dummy_d = jnp.full((16,), SPARE, jnp.int32)
    dummy_f = jnp.zeros((16,), jnp.float32)

    bufs = (bsrc, bdl, bea0, bea1)
    outs = (srcL, dstL, eaL0, eaL1)

    def flush(carry):
        wptr, off = carry
        for b, o in zip(bufs, outs):
            pltpu.sync_copy(b.at[pl.ds(0, FLUSH)],
                            o.at[pl.ds(_al8(wid * RCAP + off), FLUSH)])
        rem = wptr - FLUSH
        mrem = it < rem
        for b in bufs:
            t = plsc.load_gather(b, [FLUSH + it])
            plsc.store_scatter(b, [it], t, mask=mrem)
        return rem, off + FLUSH

    def vec_step(v, carry):
        wptr, off = carry
        s16 = plsc.load_gather(sbuf2, [v * 16 + it])
        d16 = plsc.load_gather(dbuf2, [v * 16 + it])
        a0 = plsc.load_gather(abuf0, [v * 16 + it])
        a1 = plsc.load_gather(abuf1, [v * 16 + it])
        m = (d16 >= lo) & (d16 < lo + NPT)
        mi = m.astype(jnp.int32)
        cum = plsc.cumsum(mi)
        pos = wptr + cum - mi
        pc = jnp.max(cum)
        plsc.store_scatter(bsrc, [pos], s16, mask=m)
        plsc.store_scatter(bdl, [pos], d16 - lo, mask=m)
        plsc.store_scatter(bea0, [pos], a0, mask=m)
        plsc.store_scatter(bea1, [pos], a1, mask=m)
        wptr = wptr + pc
        return lax.cond(wptr >= FLUSH, flush, lambda c: c, (wptr, off))

    def chunk(k, carry):
        pltpu.sync_copy(src_hbm.at[pl.ds(k * SCAN, SCAN)], sbuf2)
        pltpu.sync_copy(dst_hbm.at[pl.ds(k * SCAN, SCAN)], dbuf2)
        pltpu.sync_copy(ea0_hbm.at[pl.ds(k * SCAN, SCAN)], abuf0)
        pltpu.sync_copy(ea1_hbm.at[pl.ds(k * SCAN, SCAN)], abuf1)
        return lax.fori_loop(0, SCAN // 16, vec_step, carry)

    wptr, off = lax.fori_loop(0, N_EDGES // SCAN, chunk, (0, 0))

    pad1 = (-wptr) % 16
    mpad = it < pad1
    plsc.store_scatter(bsrc, [wptr + it], dummy_i, mask=mpad)
    plsc.store_scatter(bdl, [wptr + it], dummy_d, mask=mpad)
    plsc.store_scatter(bea0, [wptr + it], dummy_f, mask=mpad)
    plsc.store_scatter(bea1, [wptr + it], dummy_f, mask=mpad)
    wptr = wptr + pad1

    def padstep(_, w):
        def do(w):
            plsc.store_scatter(bsrc, [w + it], dummy_i)
            plsc.store_scatter(bdl, [w + it], dummy_d)
            plsc.store_scatter(bea0, [w + it], dummy_f)
            plsc.store_scatter(bea1, [w + it], dummy_f)
            return w + 16
        return lax.cond(w % SUP != 0, do, lambda w: w, w)

    wptr = lax.fori_loop(0, SUP // 16 - 1, padstep, wptr)

    for b, o in zip(bufs, outs):
        pltpu.sync_copy(b.at[pl.ds(0, 4096)],
                        o.at[pl.ds(_al8(wid * RCAP + off), 4096)])

    cbuf[...] = jnp.full((16,), off + wptr, jnp.int32)
    pltpu.sync_copy(cbuf, cnts2.at[pl.ds(_al8(wid * 16), 16)])


@functools.partial(
    pl.kernel,
    out_type=(
        jax.ShapeDtypeStruct((NW * RCAP,), jnp.int32),
        jax.ShapeDtypeStruct((NW * RCAP,), jnp.int32),
        jax.ShapeDtypeStruct((NW * RCAP,), jnp.float32),
        jax.ShapeDtypeStruct((NW * RCAP,), jnp.float32),
        jax.ShapeDtypeStruct((NW * 16,), jnp.int32),
    ),
    mesh=_MESH,
    compiler_params=_SC_PARAMS,
    scratch_types=[
        pltpu.VMEM((SCAN,), jnp.int32),
        pltpu.VMEM((SCAN,), jnp.int32),
        pltpu.VMEM((SCAN,), jnp.float32),
        pltpu.VMEM((SCAN,), jnp.float32),
        pltpu.VMEM((BCAP,), jnp.int32),
        pltpu.VMEM((BCAP,), jnp.int32),
        pltpu.VMEM((BCAP,), jnp.float32),
        pltpu.VMEM((BCAP,), jnp.float32),
        pltpu.VMEM((16,), jnp.int32),
        pltpu.SemaphoreType.DMA,
    ],
)
def _compact(*args):
    _compact_body(*args)


def _layer_body(xg_hbm, srcL, dstL, eaL0, eaL1, cnts2, prm_hbm, out_hbm,
                sst, dstb, a0st, a1st, xj, pv, cbuf, agg, sem):
    wid = _wid()
    it = _iota16()
    ninf = jnp.full((16,), -jnp.inf, jnp.float32)

    pltpu.sync_copy(cnts2.at[pl.ds(_al8(wid * 16), 16)], cbuf)
    cnt = jnp.max(cbuf[...])
    pltpu.sync_copy(prm_hbm, pv)

    def P(i):
        return pv[pl.ds(i * 16, 16)]

    def initr(r, _):
        rr = jnp.full((16,), r, jnp.int32)
        for c in range(8):
            plsc.store_scatter(agg, [rr, c * 16 + it], ninf)
        return 0

    lax.fori_loop(0, AGG_ROWS, initr, 0)

    def super_body(s, _):
        b0 = s * SUP
        pltpu.sync_copy(srcL.at[pl.ds(_al8(wid * RCAP + b0), SUP)], sst)
        pltpu.sync_copy(dstL.at[pl.ds(_al8(wid * RCAP + b0), SUP)], dstb)
        pltpu.sync_copy(eaL0.at[pl.ds(_al8(wid * RCAP + b0), SUP)], a0st)
        pltpu.sync_copy(eaL1.at[pl.ds(_al8(wid * RCAP + b0), SUP)], a1st)
        for cc in range(SUP // CH):
            pltpu.async_copy(
                xg_hbm.at[sst.at[pl.ds(cc * CH, CH)]], xj, sem).wait()

            def group(g2, _):
                idx16 = cc * CH + g2 * 16 + it
                dl16 = plsc.load_gather(dstb, [idx16])
                a0 = plsc.load_gather(a0st, [idx16])
                a1 = plsc.load_gather(a1st, [idx16])
                p0 = _tanh16(a0 * P(0) + a1 * P(2) + P(4))
                p1 = _tanh16(a0 * P(1) + a1 * P(3) + P(5))
                ws = []
                for k in range(K):
                    d0 = p0 - P(6 + 2 * k)
                    d1 = p1 - P(7 + 2 * k)
                    gauss = (d0 * d0 * P(12 + 2 * k)
                             + d1 * d1 * P(13 + 2 * k))
                    ws.append(jnp.exp(-0.5 * gauss))
                w0, w1, w2 = ws
                for j in range(16):
                    w0b = _lane(w0, j)
                    w1b = _lane(w1, j)
                    w2b = _lane(w2, j)
                    rowb = _lane(dl16, j)
                    er16 = jnp.full((16,), g2 * 16 + j, jnp.int32)
                    for c in range(8):
                        col = c * 16 + it
                        x0 = plsc.load_gather(xj, [er16, col])
                        x1 = plsc.load_gather(xj, [er16, 128 + col])
                        x2 = plsc.load_gather(xj, [er16, 256 + col])
                        msg = x0 * w0b + x1 * w1b + x2 * w2b
                        acc = plsc.load_gather(agg, [rowb, col])
                        plsc.store_scatter(agg, [rowb, col],
                                           jnp.maximum(acc, msg))
                return 0

            lax.fori_loop(0, CH // 16, group, 0)
        return 0

    lax.fori_loop(0, cnt // SUP, super_body, 0)
    pltpu.sync_copy(agg.at[pl.ds(0, NPT)],
                    out_hbm.at[pl.ds(_al8(wid * NPT), NPT)])


@functools.partial(
    pl.kernel,
    out_type=jax.ShapeDtypeStruct((NPAD, D), jnp.float32),
    mesh=_MESH,
    compiler_params=_SC_PARAMS,
    scratch_types=[
        pltpu.VMEM((SUP,), jnp.int32),
        pltpu.VMEM((SUP,), jnp.int32),
        pltpu.VMEM((SUP,), jnp.float32),
        pltpu.VMEM((SUP,), jnp.float32),
        pltpu.VMEM((CH, K * D), jnp.float32),
        pltpu.VMEM((NPRM * 16,), jnp.float32),
        pltpu.VMEM((16,), jnp.int32),
        pltpu.VMEM((AGG_ROWS, D), jnp.float32),
        pltpu.SemaphoreType.DMA,
    ],
)
def _sc_layer(*args):
    _layer_body(*args)


def _xg_body(h_ref, g_ref, out_ref):
    out_ref[...] = jnp.dot(h_ref[...], g_ref[...],
                           preferred_element_type=jnp.float32)


def _xg(h, g_l):
    return pl.pallas_call(
        _xg_body,
        out_shape=jax.ShapeDtypeStruct((N_NODES, K * D), jnp.float32),
    )(h, g_l)


def _xg_fused_body(a_ref, gb_ref, g_ref, out_ref):
    a = a_ref[...]
    hfix = jnp.where(a == -jnp.inf, 0.0, a) + gb_ref[...]
    h = jnp.maximum(hfix, 0.0)
    out_ref[...] = jnp.dot(h, g_ref[...], preferred_element_type=jnp.float32)


def _xg_fused(agg, gb_prev, g_l):
    return pl.pallas_call(
        _xg_fused_body,
        out_shape=jax.ShapeDtypeStruct((N_NODES, K * D), jnp.float32),
    )(agg, gb_prev.reshape(1, D), g_l)


def _head_body(a_ref, gb_ref, b_ref, fc1w_ref, fc1b_ref, fc2w_ref,
               fc2b_ref, out_ref):
    a = a_ref[...]
    h = jnp.maximum(jnp.where(a == -jnp.inf, 0.0, a) + gb_ref[...], 0.0)
    bcol = b_ref[...]
    gids = jax.lax.broadcasted_iota(jnp.int32, (N_NODES, NUM_GRAPHS), 1)
    onehot = (bcol == gids).astype(jnp.float32)
    sums = jnp.dot(onehot.T, h, preferred_element_type=jnp.float32)
    counts = jnp.sum(onehot, axis=0)
    hg = sums / jnp.clip(counts, 1.0)[:, None]
    hg = jnp.dot(hg, fc1w_ref[...], preferred_element_type=jnp.float32)
    hg = hg + fc1b_ref[...]
    hg = jnp.where(hg > 0, hg, jnp.exp(jnp.minimum(hg, 0.0)) - 1.0)
    hg = jnp.dot(hg, fc2w_ref[...], preferred_element_type=jnp.float32)
    hg = hg + fc2b_ref[...]
    m = jnp.max(hg, axis=0, keepdims=True)
    z = hg - m
    lse = jnp.log(jnp.sum(jnp.exp(z), axis=0, keepdims=True))
    out_ref[...] = z - lse


def _head(agg, gb_last, batch, fc1_w, fc1_b, fc2_w, fc2_b):
    return pl.pallas_call(
        _head_body,
        out_shape=jax.ShapeDtypeStruct((NUM_GRAPHS, OUT), jnp.float32),
    )(agg, gb_last.reshape(1, D), batch.reshape(N_NODES, 1),
      fc1_w, fc1_b.reshape(1, D), fc2_w, fc2_b.reshape(1, OUT))


def kernel(h, edge_attr, Wp, bp, g, mu, sigma, gb, fc1_w, fc1_b, fc2_w,
           fc2_b, edge_index, batch):
    src = edge_index[0]
    dst = edge_index[1]
    ea0 = edge_attr[:, 0]
    ea1 = edge_attr[:, 1]

    srcL, dstL, eaL0, eaL1, cnts2 = _compact(src, dst, ea0, ea1)

    agg = None
    for l in range(4):
        if l == 0:
            xg = _xg(h, g[0])
        else:
            xg = _xg_fused(agg, gb[l - 1], g[l])
        inv = 1.0 / (EPS + sigma[l] ** 2)
        pvec = jnp.concatenate([
            Wp[l].reshape(-1), bp[l].reshape(-1),
            mu[l].reshape(-1), inv.reshape(-1)]).astype(jnp.float32)
        prm = jnp.broadcast_to(pvec[:, None], (NPRM, 16)).reshape(-1)
        aggp = _sc_layer(xg, srcL, dstL, eaL0, eaL1, cnts2, prm)
        agg = aggp[:N_NODES]

    return _head(agg, gb[3], batch, fc1_w, fc1_b, fc2_w, fc2_b)


# speedup vs baseline: 2.6741x; 2.6741x over previous
import functools
import jax
import jax.numpy as jnp
from jax import lax
from jax.experimental import pallas as pl
from jax.experimental.pallas import tpu as pltpu
from jax.experimental.pallas import tpu_sc as plsc

N_NODES = 10000
N_EDGES = 320000
D = 128
K = 3
OUT = 10
NUM_GRAPHS = 64
EPS = 1e-15

NC = 2
NS = 16
NW = NC * NS
NPT = 320
NPAD = NW * NPT
SPARE = NPT
AGG_ROWS = NPT + 8
SCAN = 2000
FLUSH = 2048
BCAP = 4096 + 16
SUP = 512
CH = 128
RCAP = N_EDGES + 4096 + 16
NPRM = 18

_MESH = plsc.VectorSubcoreMesh(core_axis_name="c", subcore_axis_name="s")
_SC_PARAMS = pltpu.CompilerParams(needs_layout_passes=False)


def _iota16():
    return lax.iota(jnp.int32, 16)


_GDN = lax.GatherDimensionNumbers(
    offset_dims=(), collapsed_slice_dims=(0,), start_index_map=(0,))


def _lane(v, j):
    idx = jnp.full((16, 1), j, jnp.int32)
    return lax.gather(v, idx, _GDN, (1,),
                      mode=lax.GatherScatterMode.PROMISE_IN_BOUNDS)


def _wid():
    return lax.axis_index("s") * NC + lax.axis_index("c")


def _al8(x):
    return pl.multiple_of(x, 8)


def _tanh16(x):
    e = jnp.exp(jnp.clip(x + x, -40.0, 40.0))
    return (e - 1.0) / (e + 1.0)


def _compact_body(src_hbm, dst_hbm, ea0_hbm, ea1_hbm,
                  srcL, dstL, eaL0, eaL1, cnts2,
                  sbuf2, dbuf2, abuf0, abuf1,
                  bsrc, bdl, bea0, bea1, cbuf, sem):
    wid = _wid()
    lo = wid * NPT
    it = _iota16()
    dummy_i = jnp.zeros((16,), jnp.int32)
    dummy_d = jnp.full((16,), SPARE, jnp.int32)
    dummy_f = jnp.zeros((16,), jnp.float32)

    bufs = (bsrc, bdl, bea0, bea1)
    outs = (srcL, dstL, eaL0, eaL1)

    def flush(carry):
        wptr, off = carry
        for b, o in zip(bufs, outs):
            pltpu.sync_copy(b.at[pl.ds(0, FLUSH)],
                            o.at[pl.ds(_al8(wid * RCAP + off), FLUSH)])
        rem = wptr - FLUSH
        mrem = it < rem
        for b in bufs:
            t = plsc.load_gather(b, [FLUSH + it])
            plsc.store_scatter(b, [it], t, mask=mrem)
        return rem, off + FLUSH

    def vec_step(v, carry):
        wptr, off = carry
        s16 = plsc.load_gather(sbuf2, [v * 16 + it])
        d16 = plsc.load_gather(dbuf2, [v * 16 + it])
        a0 = plsc.load_gather(abuf0, [v * 16 + it])
        a1 = plsc.load_gather(abuf1, [v * 16 + it])
        m = (d16 >= lo) & (d16 < lo + NPT)
        mi = m.astype(jnp.int32)
        cum = plsc.cumsum(mi)
        pos = wptr + cum - mi
        pc = jnp.max(cum)
        plsc.store_scatter(bsrc, [pos], s16, mask=m)
        plsc.store_scatter(bdl, [pos], d16 - lo, mask=m)
        plsc.store_scatter(bea0, [pos], a0, mask=m)
        plsc.store_scatter(bea1, [pos], a1, mask=m)
        wptr = wptr + pc
        return lax.cond(wptr >= FLUSH, flush, lambda c: c, (wptr, off))

    def chunk(k, carry):
        pltpu.sync_copy(src_hbm.at[pl.ds(k * SCAN, SCAN)], sbuf2)
        pltpu.sync_copy(dst_hbm.at[pl.ds(k * SCAN, SCAN)], dbuf2)
        pltpu.sync_copy(ea0_hbm.at[pl.ds(k * SCAN, SCAN)], abuf0)
        pltpu.sync_copy(ea1_hbm.at[pl.ds(k * SCAN, SCAN)], abuf1)
        return lax.fori_loop(0, SCAN // 16, vec_step, carry)

    wptr, off = lax.fori_loop(0, N_EDGES // SCAN, chunk, (0, 0))

    pad1 = (-wptr) % 16
    mpad = it < pad1
    plsc.store_scatter(bsrc, [wptr + it], dummy_i, mask=mpad)
    plsc.store_scatter(bdl, [wptr + it], dummy_d, mask=mpad)
    plsc.store_scatter(bea0, [wptr + it], dummy_f, mask=mpad)
    plsc.store_scatter(bea1, [wptr + it], dummy_f, mask=mpad)
    wptr = wptr + pad1

    def padstep(_, w):
        def do(w):
            plsc.store_scatter(bsrc, [w + it], dummy_i)
            plsc.store_scatter(bdl, [w + it], dummy_d)
            plsc.store_scatter(bea0, [w + it], dummy_f)
            plsc.store_scatter(bea1, [w + it], dummy_f)
            return w + 16
        return lax.cond(w % SUP != 0, do, lambda w: w, w)

    wptr = lax.fori_loop(0, SUP // 16 - 1, padstep, wptr)

    for b, o in zip(bufs, outs):
        pltpu.sync_copy(b.at[pl.ds(0, 4096)],
                        o.at[pl.ds(_al8(wid * RCAP + off), 4096)])

    cbuf[...] = jnp.full((16,), off + wptr, jnp.int32)
    pltpu.sync_copy(cbuf, cnts2.at[pl.ds(_al8(wid * 16), 16)])


@functools.partial(
    pl.kernel,
    out_type=(
        jax.ShapeDtypeStruct((NW * RCAP,), jnp.int32),
        jax.ShapeDtypeStruct((NW * RCAP,), jnp.int32),
        jax.ShapeDtypeStruct((NW * RCAP,), jnp.float32),
        jax.ShapeDtypeStruct((NW * RCAP,), jnp.float32),
        jax.ShapeDtypeStruct((NW * 16,), jnp.int32),
    ),
    mesh=_MESH,
    compiler_params=_SC_PARAMS,
    scratch_types=[
        pltpu.VMEM((SCAN,), jnp.int32),
        pltpu.VMEM((SCAN,), jnp.int32),
        pltpu.VMEM((SCAN,), jnp.float32),
        pltpu.VMEM((SCAN,), jnp.float32),
        pltpu.VMEM((BCAP,), jnp.int32),
        pltpu.VMEM((BCAP,), jnp.int32),
        pltpu.VMEM((BCAP,), jnp.float32),
        pltpu.VMEM((BCAP,), jnp.float32),
        pltpu.VMEM((16,), jnp.int32),
        pltpu.SemaphoreType.DMA,
    ],
)
def _compact(*args):
    _compact_body(*args)


def _layer_body(xg_hbm, srcL, dstL, eaL0, eaL1, cnts2, prm_hbm, out_hbm,
                sst, dstb, a0st, a1st, xj, pv, cbuf, agg, sem):
    wid = _wid()
    it = _iota16()
    ninf = jnp.full((16,), -jnp.inf, jnp.float32)

    pltpu.sync_copy(cnts2.at[pl.ds(_al8(wid * 16), 16)], cbuf)
    cnt = jnp.max(cbuf[...])
    pltpu.sync_copy(prm_hbm, pv)

    def P(i):
        return pv[pl.ds(i * 16, 16)]

    def initr(r, _):
        rr = jnp.full((16,), r, jnp.int32)
        for c in range(8):
            plsc.store_scatter(agg, [rr, c * 16 + it], ninf)
        return 0

    lax.fori_loop(0, AGG_ROWS, initr, 0)

    def super_body(s, _):
        b0 = s * SUP
        pltpu.sync_copy(srcL.at[pl.ds(_al8(wid * RCAP + b0), SUP)], sst)
        pltpu.sync_copy(dstL.at[pl.ds(_al8(wid * RCAP + b0), SUP)], dstb)
        pltpu.sync_copy(eaL0.at[pl.ds(_al8(wid * RCAP + b0), SUP)], a0st)
        pltpu.sync_copy(eaL1.at[pl.ds(_al8(wid * RCAP + b0), SUP)], a1st)
        for cc in range(SUP // CH):
            pltpu.async_copy(
                xg_hbm.at[sst.at[pl.ds(cc * CH, CH)]], xj, sem).wait()

            def group(g2, _):
                idx16 = cc * CH + g2 * 16 + it
                dl16 = plsc.load_gather(dstb, [idx16])
                a0 = plsc.load_gather(a0st, [idx16])
                a1 = plsc.load_gather(a1st, [idx16])
                p0 = _tanh16(a0 * P(0) + a1 * P(2) + P(4))
                p1 = _tanh16(a0 * P(1) + a1 * P(3) + P(5))
                ws = []
                for k in range(K):
                    d0 = p0 - P(6 + 2 * k)
                    d1 = p1 - P(7 + 2 * k)
                    gauss = (d0 * d0 * P(12 + 2 * k)
                             + d1 * d1 * P(13 + 2 * k))
                    ws.append(jnp.exp(-0.5 * gauss))
                w0, w1, w2 = ws
                for j in range(16):
                    w0b = _lane(w0, j)
                    w1b = _lane(w1, j)
                    w2b = _lane(w2, j)
                    rowb = _lane(dl16, j)
                    er16 = jnp.full((16,), g2 * 16 + j, jnp.int32)
                    msgs = []
                    for c in range(8):
                        col = c * 16 + it
                        x0 = plsc.load_gather(xj, [er16, col])
                        x1 = plsc.load_gather(xj, [er16, 128 + col])
                        x2 = plsc.load_gather(xj, [er16, 256 + col])
                        msgs.append(x0 * w0b + x1 * w1b + x2 * w2b)
                    accs = [plsc.load_gather(agg, [rowb, c * 16 + it])
                            for c in range(8)]
                    for c in range(8):
                        plsc.store_scatter(agg, [rowb, c * 16 + it],
                                           jnp.maximum(accs[c], msgs[c]))
                return 0

            lax.fori_loop(0, CH // 16, group, 0)
        return 0

    lax.fori_loop(0, cnt // SUP, super_body, 0)
    pltpu.sync_copy(agg.at[pl.ds(0, NPT)],
                    out_hbm.at[pl.ds(_al8(wid * NPT), NPT)])


@functools.partial(
    pl.kernel,
    out_type=jax.ShapeDtypeStruct((NPAD, D), jnp.float32),
    mesh=_MESH,
    compiler_params=_SC_PARAMS,
    scratch_types=[
        pltpu.VMEM((SUP,), jnp.int32),
        pltpu.VMEM((SUP,), jnp.int32),
        pltpu.VMEM((SUP,), jnp.float32),
        pltpu.VMEM((SUP,), jnp.float32),
        pltpu.VMEM((CH, K * D), jnp.float32),
        pltpu.VMEM((NPRM * 16,), jnp.float32),
        pltpu.VMEM((16,), jnp.int32),
        pltpu.VMEM((AGG_ROWS, D), jnp.float32),
        pltpu.SemaphoreType.DMA,
    ],
)
def _sc_layer(*args):
    _layer_body(*args)


def _xg_body(h_ref, g_ref, out_ref):
    out_ref[...] = jnp.dot(h_ref[...], g_ref[...],
                           preferred_element_type=jnp.float32)


def _xg(h, g_l):
    return pl.pallas_call(
        _xg_body,
        out_shape=jax.ShapeDtypeStruct((N_NODES, K * D), jnp.float32),
    )(h, g_l)


def _xg_fused_body(a_ref, gb_ref, g_ref, out_ref):
    a = a_ref[...]
    hfix = jnp.where(a == -jnp.inf, 0.0, a) + gb_ref[...]
    h = jnp.maximum(hfix, 0.0)
    out_ref[...] = jnp.dot(h, g_ref[...], preferred_element_type=jnp.float32)


def _xg_fused(agg, gb_prev, g_l):
    return pl.pallas_call(
        _xg_fused_body,
        out_shape=jax.ShapeDtypeStruct((N_NODES, K * D), jnp.float32),
    )(agg, gb_prev.reshape(1, D), g_l)


def _head_body(a_ref, gb_ref, b_ref, fc1w_ref, fc1b_ref, fc2w_ref,
               fc2b_ref, out_ref):
    a = a_ref[...]
    h = jnp.maximum(jnp.where(a == -jnp.inf, 0.0, a) + gb_ref[...], 0.0)
    bcol = b_ref[...]
    gids = jax.lax.broadcasted_iota(jnp.int32, (N_NODES, NUM_GRAPHS), 1)
    onehot = (bcol == gids).astype(jnp.float32)
    sums = jnp.dot(onehot.T, h, preferred_element_type=jnp.float32)
    counts = jnp.sum(onehot, axis=0)
    hg = sums / jnp.clip(counts, 1.0)[:, None]
    hg = jnp.dot(hg, fc1w_ref[...], preferred_element_type=jnp.float32)
    hg = hg + fc1b_ref[...]
    hg = jnp.where(hg > 0, hg, jnp.exp(jnp.minimum(hg, 0.0)) - 1.0)
    hg = jnp.dot(hg, fc2w_ref[...], preferred_element_type=jnp.float32)
    hg = hg + fc2b_ref[...]
    m = jnp.max(hg, axis=0, keepdims=True)
    z = hg - m
    lse = jnp.log(jnp.sum(jnp.exp(z), axis=0, keepdims=True))
    out_ref[...] = z - lse


def _head(agg, gb_last, batch, fc1_w, fc1_b, fc2_w, fc2_b):
    return pl.pallas_call(
        _head_body,
        out_shape=jax.ShapeDtypeStruct((NUM_GRAPHS, OUT), jnp.float32),
    )(agg, gb_last.reshape(1, D), batch.reshape(N_NODES, 1),
      fc1_w, fc1_b.reshape(1, D), fc2_w, fc2_b.reshape(1, OUT))


def kernel(h, edge_attr, Wp, bp, g, mu, sigma, gb, fc1_w, fc1_b, fc2_w,
           fc2_b, edge_index, batch):
    src = edge_index[0]
    dst = edge_index[1]
    ea0 = edge_attr[:, 0]
    ea1 = edge_attr[:, 1]

    srcL, dstL, eaL0, eaL1, cnts2 = _compact(src, dst, ea0, ea1)

    agg = None
    for l in range(4):
        if l == 0:
            xg = _xg(h, g[0])
        else:
            xg = _xg_fused(agg, gb[l - 1], g[l])
        inv = 1.0 / (EPS + sigma[l] ** 2)
        pvec = jnp.concatenate([
            Wp[l].reshape(-1), bp[l].reshape(-1),
            mu[l].reshape(-1), inv.reshape(-1)]).astype(jnp.float32)
        prm = jnp.broadcast_to(pvec[:, None], (NPRM, 16)).reshape(-1)
        aggp = _sc_layer(xg, srcL, dstL, eaL0, eaL1, cnts2, prm)
        agg = aggp[:N_NODES]

    return _head(agg, gb[3], batch, fc1_w, fc1_b, fc2_w, fc2_b)


# speedup vs baseline: 3.3674x; 1.2593x over previous
import functools
import jax
import jax.numpy as jnp
from jax import lax
from jax.experimental import pallas as pl
from jax.experimental.pallas import tpu as pltpu
from jax.experimental.pallas import tpu_sc as plsc

N_NODES = 10000
N_EDGES = 320000
D = 128
K = 3
OUT = 10
NUM_GRAPHS = 64
EPS = 1e-15

NC = 2
NS = 16
NW = NC * NS
NPT = 320
NPAD = NW * NPT
SPARE = NPT
AGG_ROWS = NPT + 8
SCAN = 2000
FLUSH = 2048
BCAP = FLUSH + SCAN + 2048
SUP = 512
CH = 64
RCAP = N_EDGES + 4096 + 16
NPRM = 18

_MESH = plsc.VectorSubcoreMesh(core_axis_name="c", subcore_axis_name="s")
_SC_PARAMS = pltpu.CompilerParams(needs_layout_passes=False)


def _iota16():
    return lax.iota(jnp.int32, 16)


_GDN = lax.GatherDimensionNumbers(
    offset_dims=(), collapsed_slice_dims=(0,), start_index_map=(0,))


def _lane(v, j):
    idx = jnp.full((16, 1), j, jnp.int32)
    return lax.gather(v, idx, _GDN, (1,),
                      mode=lax.GatherScatterMode.PROMISE_IN_BOUNDS)


def _wid():
    return lax.axis_index("s") * NC + lax.axis_index("c")


def _al8(x):
    return pl.multiple_of(x, 8)


def _tanh16(x):
    e = jnp.exp(jnp.clip(x + x, -40.0, 40.0))
    return (e - 1.0) / (e + 1.0)


def _compact_body(src_hbm, dst_hbm, ea0_hbm, ea1_hbm,
                  srcL, dstL, eaL0, eaL1, cnts2,
                  sbuf2, dbuf2, abuf0, abuf1,
                  bsrc, bdl, bea0, bea1, cbuf, sem):
    wid = _wid()
    lo = wid * NPT
    it = _iota16()
    dummy_i = jnp.zeros((16,), jnp.int32)
    dummy_d = jnp.full((16,), SPARE, jnp.int32)
    dummy_f = jnp.zeros((16,), jnp.float32)

    bufs = (bsrc, bdl, bea0, bea1)
    outs = (srcL, dstL, eaL0, eaL1)

    def flush(carry):
        wptr, off = carry
        for b, o in zip(bufs, outs):
            pltpu.sync_copy(b.at[pl.ds(0, FLUSH)],
                            o.at[pl.ds(_al8(wid * RCAP + off), FLUSH)])
        rem = wptr - FLUSH
        mrem = it < rem
        for b in bufs:
            t = plsc.load_gather(b, [FLUSH + it])
            plsc.store_scatter(b, [it], t, mask=mrem)
        return rem, off + FLUSH

    def vec_step(v, carry):
        wptr, off = carry
        s16 = plsc.load_gather(sbuf2, [v * 16 + it])
        d16 = plsc.load_gather(dbuf2, [v * 16 + it])
        a0 = plsc.load_gather(abuf0, [v * 16 + it])
        a1 = plsc.load_gather(abuf1, [v * 16 + it])
        m = (d16 >= lo) & (d16 < lo + NPT)
        mi = m.astype(jnp.int32)
        cum = plsc.cumsum(mi)
        pos = wptr + cum - mi
        pc = jnp.max(cum)
        plsc.store_scatter(bsrc, [pos], s16, mask=m)
        plsc.store_scatter(bdl, [pos], d16 - lo, mask=m)
        plsc.store_scatter(bea0, [pos], a0, mask=m)
        plsc.store_scatter(bea1, [pos], a1, mask=m)
        return wptr + pc, off

    def chunk(k, carry):
        pltpu.sync_copy(src_hbm.at[pl.ds(k * SCAN, SCAN)], sbuf2)
        pltpu.sync_copy(dst_hbm.at[pl.ds(k * SCAN, SCAN)], dbuf2)
        pltpu.sync_copy(ea0_hbm.at[pl.ds(k * SCAN, SCAN)], abuf0)
        pltpu.sync_copy(ea1_hbm.at[pl.ds(k * SCAN, SCAN)], abuf1)
        carry = lax.fori_loop(0, SCAN // 16, vec_step, carry)
        return lax.cond(carry[0] >= FLUSH, flush, lambda c: c, carry)

    wptr, off = lax.fori_loop(0, N_EDGES // SCAN, chunk, (0, 0))

    pad1 = (-wptr) % 16
    mpad = it < pad1
    plsc.store_scatter(bsrc, [wptr + it], dummy_i, mask=mpad)
    plsc.store_scatter(bdl, [wptr + it], dummy_d, mask=mpad)
    plsc.store_scatter(bea0, [wptr + it], dummy_f, mask=mpad)
    plsc.store_scatter(bea1, [wptr + it], dummy_f, mask=mpad)
    wptr = wptr + pad1

    def padstep(_, w):
        def do(w):
            plsc.store_scatter(bsrc, [w + it], dummy_i)
            plsc.store_scatter(bdl, [w + it], dummy_d)
            plsc.store_scatter(bea0, [w + it], dummy_f)
            plsc.store_scatter(bea1, [w + it], dummy_f)
            return w + 16
        return lax.cond(w % SUP != 0, do, lambda w: w, w)

    wptr = lax.fori_loop(0, SUP // 16 - 1, padstep, wptr)

    for b, o in zip(bufs, outs):
        pltpu.sync_copy(b.at[pl.ds(0, 4096)],
                        o.at[pl.ds(_al8(wid * RCAP + off), 4096)])

    cbuf[...] = jnp.full((16,), off + wptr, jnp.int32)
    pltpu.sync_copy(cbuf, cnts2.at[pl.ds(_al8(wid * 16), 16)])


@functools.partial(
    pl.kernel,
    out_type=(
        jax.ShapeDtypeStruct((NW * RCAP,), jnp.int32),
        jax.ShapeDtypeStruct((NW * RCAP,), jnp.int32),
        jax.ShapeDtypeStruct((NW * RCAP,), jnp.float32),
        jax.ShapeDtypeStruct((NW * RCAP,), jnp.float32),
        jax.ShapeDtypeStruct((NW * 16,), jnp.int32),
    ),
    mesh=_MESH,
    compiler_params=_SC_PARAMS,
    scratch_types=[
        pltpu.VMEM((SCAN,), jnp.int32),
        pltpu.VMEM((SCAN,), jnp.int32),
        pltpu.VMEM((SCAN,), jnp.float32),
        pltpu.VMEM((SCAN,), jnp.float32),
        pltpu.VMEM((BCAP,), jnp.int32),
        pltpu.VMEM((BCAP,), jnp.int32),
        pltpu.VMEM((BCAP,), jnp.float32),
        pltpu.VMEM((BCAP,), jnp.float32),
        pltpu.VMEM((16,), jnp.int32),
        pltpu.SemaphoreType.DMA,
    ],
)
def _compact(*args):
    _compact_body(*args)


def _layer_body(xg_hbm, srcL, dstL, eaL0, eaL1, cnts2, prm_hbm, out_hbm,
                sst, dstb, a0st, a1st, xja, xjb, pv, cbuf, agg, sema, semb):
    sems = (sema, semb)
    wid = _wid()
    it = _iota16()
    ninf = jnp.full((16,), -jnp.inf, jnp.float32)

    pltpu.sync_copy(cnts2.at[pl.ds(_al8(wid * 16), 16)], cbuf)
    cnt = jnp.max(cbuf[...])
    pltpu.sync_copy(prm_hbm, pv)

    def P(i):
        return pv[pl.ds(i * 16, 16)]

    def initr(r, _):
        rr = jnp.full((16,), r, jnp.int32)
        for c in range(8):
            plsc.store_scatter(agg, [rr, c * 16 + it], ninf)
        return 0

    lax.fori_loop(0, AGG_ROWS, initr, 0)

    def super_body(s, _):
        b0 = s * SUP
        pltpu.sync_copy(srcL.at[pl.ds(_al8(wid * RCAP + b0), SUP)], sst)
        pltpu.sync_copy(dstL.at[pl.ds(_al8(wid * RCAP + b0), SUP)], dstb)
        pltpu.sync_copy(eaL0.at[pl.ds(_al8(wid * RCAP + b0), SUP)], a0st)
        pltpu.sync_copy(eaL1.at[pl.ds(_al8(wid * RCAP + b0), SUP)], a1st)
        nch = SUP // CH
        hnd = pltpu.async_copy(
            xg_hbm.at[sst.at[pl.ds(0, CH)]], xja, sems[0])
        for cc in range(nch):
            xj = (xja, xjb)[cc % 2]
            hnd.wait()
            if cc + 1 < nch:
                hnd = pltpu.async_copy(
                    xg_hbm.at[sst.at[pl.ds((cc + 1) * CH, CH)]],
                    (xja, xjb)[(cc + 1) % 2], sems[(cc + 1) % 2])

            def group(g2, _):
                idx16 = cc * CH + g2 * 16 + it
                dl16 = plsc.load_gather(dstb, [idx16])
                a0 = plsc.load_gather(a0st, [idx16])
                a1 = plsc.load_gather(a1st, [idx16])
                p0 = _tanh16(a0 * P(0) + a1 * P(2) + P(4))
                p1 = _tanh16(a0 * P(1) + a1 * P(3) + P(5))
                ws = []
                for k in range(K):
                    d0 = p0 - P(6 + 2 * k)
                    d1 = p1 - P(7 + 2 * k)
                    gauss = (d0 * d0 * P(12 + 2 * k)
                             + d1 * d1 * P(13 + 2 * k))
                    ws.append(jnp.exp(-0.5 * gauss))
                w0, w1, w2 = ws
                for j in range(16):
                    w0b = _lane(w0, j)
                    w1b = _lane(w1, j)
                    w2b = _lane(w2, j)
                    rowb = _lane(dl16, j)
                    er16 = jnp.full((16,), g2 * 16 + j, jnp.int32)
                    msgs = []
                    for c in range(8):
                        col = c * 16 + it
                        x0 = plsc.load_gather(xj, [er16, col])
                        x1 = plsc.load_gather(xj, [er16, 128 + col])
                        x2 = plsc.load_gather(xj, [er16, 256 + col])
                        msgs.append(x0 * w0b + x1 * w1b + x2 * w2b)
                    accs = [plsc.load_gather(agg, [rowb, c * 16 + it])
                            for c in range(8)]
                    for c in range(8):
                        plsc.store_scatter(agg, [rowb, c * 16 + it],
                                           jnp.maximum(accs[c], msgs[c]))
                return 0

            lax.fori_loop(0, CH // 16, group, 0)
        return 0

    lax.fori_loop(0, cnt // SUP, super_body, 0)
    pltpu.sync_copy(agg.at[pl.ds(0, NPT)],
                    out_hbm.at[pl.ds(_al8(wid * NPT), NPT)])


@functools.partial(
    pl.kernel,
    out_type=jax.ShapeDtypeStruct((NPAD, D), jnp.float32),
    mesh=_MESH,
    compiler_params=_SC_PARAMS,
    scratch_types=[
        pltpu.VMEM((SUP,), jnp.int32),
        pltpu.VMEM((SUP,), jnp.int32),
        pltpu.VMEM((SUP,), jnp.float32),
        pltpu.VMEM((SUP,), jnp.float32),
        pltpu.VMEM((CH, K * D), jnp.float32),
        pltpu.VMEM((CH, K * D), jnp.float32),
        pltpu.VMEM((NPRM * 16,), jnp.float32),
        pltpu.VMEM((16,), jnp.int32),
        pltpu.VMEM((AGG_ROWS, D), jnp.float32),
        pltpu.SemaphoreType.DMA,
        pltpu.SemaphoreType.DMA,
    ],
)
def _sc_layer(*args):
    _layer_body(*args)


def _xg_body(h_ref, g_ref, out_ref):
    out_ref[...] = jnp.dot(h_ref[...], g_ref[...],
                           preferred_element_type=jnp.float32)


def _xg(h, g_l):
    return pl.pallas_call(
        _xg_body,
        out_shape=jax.ShapeDtypeStruct((N_NODES, K * D), jnp.float32),
    )(h, g_l)


def _xg_fused_body(a_ref, gb_ref, g_ref, out_ref):
    a = a_ref[...]
    hfix = jnp.where(a == -jnp.inf, 0.0, a) + gb_ref[...]
    h = jnp.maximum(hfix, 0.0)
    out_ref[...] = jnp.dot(h, g_ref[...], preferred_element_type=jnp.float32)


def _xg_fused(agg, gb_prev, g_l):
    return pl.pallas_call(
        _xg_fused_body,
        out_shape=jax.ShapeDtypeStruct((N_NODES, K * D), jnp.float32),
    )(agg, gb_prev.reshape(1, D), g_l)


def _head_body(a_ref, gb_ref, b_ref, fc1w_ref, fc1b_ref, fc2w_ref,
               fc2b_ref, out_ref):
    a = a_ref[...]
    h = jnp.maximum(jnp.where(a == -jnp.inf, 0.0, a) + gb_ref[...], 0.0)
    bcol = b_ref[...]
    gids = jax.lax.broadcasted_iota(jnp.int32, (N_NODES, NUM_GRAPHS), 1)
    onehot = (bcol == gids).astype(jnp.float32)
    sums = jnp.dot(onehot.T, h, preferred_element_type=jnp.float32)
    counts = jnp.sum(onehot, axis=0)
    hg = sums / jnp.clip(counts, 1.0)[:, None]
    hg = jnp.dot(hg, fc1w_ref[...], preferred_element_type=jnp.float32)
    hg = hg + fc1b_ref[...]
    hg = jnp.where(hg > 0, hg, jnp.exp(jnp.minimum(hg, 0.0)) - 1.0)
    hg = jnp.dot(hg, fc2w_ref[...], preferred_element_type=jnp.float32)
    hg = hg + fc2b_ref[...]
    m = jnp.max(hg, axis=0, keepdims=True)
    z = hg - m
    lse = jnp.log(jnp.sum(jnp.exp(z), axis=0, keepdims=True))
    out_ref[...] = z - lse


def _head(agg, gb_last, batch, fc1_w, fc1_b, fc2_w, fc2_b):
    return pl.pallas_call(
        _head_body,
        out_shape=jax.ShapeDtypeStruct((NUM_GRAPHS, OUT), jnp.float32),
    )(agg, gb_last.reshape(1, D), batch.reshape(N_NODES, 1),
      fc1_w, fc1_b.reshape(1, D), fc2_w, fc2_b.reshape(1, OUT))


def kernel(h, edge_attr, Wp, bp, g, mu, sigma, gb, fc1_w, fc1_b, fc2_w,
           fc2_b, edge_index, batch):
    src = edge_index[0]
    dst = edge_index[1]
    ea0 = edge_attr[:, 0]
    ea1 = edge_attr[:, 1]

    srcL, dstL, eaL0, eaL1, cnts2 = _compact(src, dst, ea0, ea1)

    agg = None
    for l in range(4):
        if l == 0:
            xg = _xg(h, g[0])
        else:
            xg = _xg_fused(agg, gb[l - 1], g[l])
        inv = 1.0 / (EPS + sigma[l] ** 2)
        pvec = jnp.concatenate([
            Wp[l].reshape(-1), bp[l].reshape(-1),
            mu[l].reshape(-1), inv.reshape(-1)]).astype(jnp.float32)
        prm = jnp.broadcast_to(pvec[:, None], (NPRM, 16)).reshape(-1)
        aggp = _sc_layer(xg, srcL, dstL, eaL0, eaL1, cnts2, prm)
        agg = aggp[:N_NODES]

    return _head(agg, gb[3], batch, fc1_w, fc1_b, fc2_w, fc2_b)


# speedup vs baseline: 3.7314x; 1.1081x over previous
import functools
import jax
import jax.numpy as jnp
from jax import lax
from jax.experimental import pallas as pl
from jax.experimental.pallas import tpu as pltpu
from jax.experimental.pallas import tpu_sc as plsc

N_NODES = 10000
N_EDGES = 320000
D = 128
K = 3
OUT = 10
NUM_GRAPHS = 64
EPS = 1e-15

NC = 2
NS = 16
NW = NC * NS
NPT = 320
NPAD = NW * NPT
SPARE = NPT
AGG_ROWS = NPT + 8
SCAN = 2000
FLUSH = 2048
BCAP = FLUSH + SCAN + 2048
SUP = 512
CH = 64
RCAP = N_EDGES + 4096 + 16
NPRM = 18

_MESH = plsc.VectorSubcoreMesh(core_axis_name="c", subcore_axis_name="s")
_SC_PARAMS = pltpu.CompilerParams(needs_layout_passes=False)


def _iota16():
    return lax.iota(jnp.int32, 16)


_GDN = lax.GatherDimensionNumbers(
    offset_dims=(), collapsed_slice_dims=(0,), start_index_map=(0,))


def _lane(v, j):
    idx = jnp.full((16, 1), j, jnp.int32)
    return lax.gather(v, idx, _GDN, (1,),
                      mode=lax.GatherScatterMode.PROMISE_IN_BOUNDS)


def _wid():
    return lax.axis_index("s") * NC + lax.axis_index("c")


def _al8(x):
    return pl.multiple_of(x, 8)


def _tanh16(x):
    e = jnp.exp(jnp.clip(x + x, -40.0, 40.0))
    return (e - 1.0) / (e + 1.0)


def _compact_body(src_hbm, dst_hbm, ea0_hbm, ea1_hbm,
                  srcL, dstL, eaL0, eaL1, cnts2,
                  sbuf2, dbuf2, abuf0, abuf1,
                  bsrc, bdl, bea0, bea1, cbuf, sem):
    wid = _wid()
    lo = wid * NPT
    it = _iota16()
    dummy_i = jnp.zeros((16,), jnp.int32)
    dummy_d = jnp.full((16,), SPARE, jnp.int32)
    dummy_f = jnp.zeros((16,), jnp.float32)

    bufs = (bsrc, bdl, bea0, bea1)
    outs = (srcL, dstL, eaL0, eaL1)

    def flush(carry):
        wptr, off = carry
        for b, o in zip(bufs, outs):
            pltpu.sync_copy(b.at[pl.ds(0, FLUSH)],
                            o.at[pl.ds(_al8(wid * RCAP + off), FLUSH)])
        rem = wptr - FLUSH
        mrem = it < rem
        for b in bufs:
            t = plsc.load_gather(b, [FLUSH + it])
            plsc.store_scatter(b, [it], t, mask=mrem)
        return rem, off + FLUSH

    def vec_step(v, carry):
        wptr, off = carry
        vals = []
        for h in range(2):
            b = v * 32 + h * 16 + it
            s16 = plsc.load_gather(sbuf2, [b])
            d16 = plsc.load_gather(dbuf2, [b])
            a0 = plsc.load_gather(abuf0, [b])
            a1 = plsc.load_gather(abuf1, [b])
            m = (d16 >= lo) & (d16 < lo + NPT)
            mi = m.astype(jnp.int32)
            cum = plsc.cumsum(mi)
            vals.append((s16, d16, a0, a1, m, mi, cum))
        base = wptr
        for s16, d16, a0, a1, m, mi, cum in vals:
            pos = base + cum - mi
            plsc.store_scatter(bsrc, [pos], s16, mask=m)
            plsc.store_scatter(bdl, [pos], d16 - lo, mask=m)
            plsc.store_scatter(bea0, [pos], a0, mask=m)
            plsc.store_scatter(bea1, [pos], a1, mask=m)
            base = base + _lane(cum, 15)
        return base, off

    def chunk(k, carry):
        pltpu.sync_copy(src_hbm.at[pl.ds(k * SCAN, SCAN)], sbuf2)
        pltpu.sync_copy(dst_hbm.at[pl.ds(k * SCAN, SCAN)], dbuf2)
        pltpu.sync_copy(ea0_hbm.at[pl.ds(k * SCAN, SCAN)], abuf0)
        pltpu.sync_copy(ea1_hbm.at[pl.ds(k * SCAN, SCAN)], abuf1)
        carry = lax.fori_loop(0, SCAN // 32, vec_step, carry)
        return lax.cond(jnp.max(carry[0]) >= FLUSH, flush, lambda c: c,
                        carry)

    wptr, off = lax.fori_loop(0, N_EDGES // SCAN, chunk,
                              (jnp.zeros((16,), jnp.int32), 0))

    pad1 = (-wptr) % 16
    mpad = it < pad1
    plsc.store_scatter(bsrc, [wptr + it], dummy_i, mask=mpad)
    plsc.store_scatter(bdl, [wptr + it], dummy_d, mask=mpad)
    plsc.store_scatter(bea0, [wptr + it], dummy_f, mask=mpad)
    plsc.store_scatter(bea1, [wptr + it], dummy_f, mask=mpad)
    wptr = wptr + pad1

    def padstep(_, w):
        def do(w):
            plsc.store_scatter(bsrc, [w + it], dummy_i)
            plsc.store_scatter(bdl, [w + it], dummy_d)
            plsc.store_scatter(bea0, [w + it], dummy_f)
            plsc.store_scatter(bea1, [w + it], dummy_f)
            return w + 16
        return lax.cond(jnp.max(w % SUP) != 0, do, lambda w: w, w)

    wptr = lax.fori_loop(0, SUP // 16 - 1, padstep, wptr)

    for b, o in zip(bufs, outs):
        pltpu.sync_copy(b.at[pl.ds(0, 4096)],
                        o.at[pl.ds(_al8(wid * RCAP + off), 4096)])

    cbuf[...] = (off + wptr).astype(jnp.int32)
    pltpu.sync_copy(cbuf, cnts2.at[pl.ds(_al8(wid * 16), 16)])


@functools.partial(
    pl.kernel,
    out_type=(
        jax.ShapeDtypeStruct((NW * RCAP,), jnp.int32),
        jax.ShapeDtypeStruct((NW * RCAP,), jnp.int32),
        jax.ShapeDtypeStruct((NW * RCAP,), jnp.float32),
        jax.ShapeDtypeStruct((NW * RCAP,), jnp.float32),
        jax.ShapeDtypeStruct((NW * 16,), jnp.int32),
    ),
    mesh=_MESH,
    compiler_params=_SC_PARAMS,
    scratch_types=[
        pltpu.VMEM((SCAN,), jnp.int32),
        pltpu.VMEM((SCAN,), jnp.int32),
        pltpu.VMEM((SCAN,), jnp.float32),
        pltpu.VMEM((SCAN,), jnp.float32),
        pltpu.VMEM((BCAP,), jnp.int32),
        pltpu.VMEM((BCAP,), jnp.int32),
        pltpu.VMEM((BCAP,), jnp.float32),
        pltpu.VMEM((BCAP,), jnp.float32),
        pltpu.VMEM((16,), jnp.int32),
        pltpu.SemaphoreType.DMA,
    ],
)
def _compact(*args):
    _compact_body(*args)


def _layer_body(xg_hbm, srcL, dstL, eaL0, eaL1, cnts2, prm_hbm, out_hbm,
                sst, dstb, a0st, a1st, xja, xjb, pv, cbuf, agg, sema, semb):
    sems = (sema, semb)
    wid = _wid()
    it = _iota16()
    ninf = jnp.full((16,), -jnp.inf, jnp.float32)

    pltpu.sync_copy(cnts2.at[pl.ds(_al8(wid * 16), 16)], cbuf)
    cnt = jnp.max(cbuf[...])
    pltpu.sync_copy(prm_hbm, pv)

    def P(i):
        return pv[pl.ds(i * 16, 16)]

    def initr(r, _):
        rr = jnp.full((16,), r, jnp.int32)
        for c in range(8):
            plsc.store_scatter(agg, [rr, c * 16 + it], ninf)
        return 0

    lax.fori_loop(0, AGG_ROWS, initr, 0)

    def super_body(s, _):
        b0 = s * SUP
        pltpu.sync_copy(srcL.at[pl.ds(_al8(wid * RCAP + b0), SUP)], sst)
        pltpu.sync_copy(dstL.at[pl.ds(_al8(wid * RCAP + b0), SUP)], dstb)
        pltpu.sync_copy(eaL0.at[pl.ds(_al8(wid * RCAP + b0), SUP)], a0st)
        pltpu.sync_copy(eaL1.at[pl.ds(_al8(wid * RCAP + b0), SUP)], a1st)
        nch = SUP // CH
        hnd = pltpu.async_copy(
            xg_hbm.at[sst.at[pl.ds(0, CH)]], xja, sems[0])
        for cc in range(nch):
            xj = (xja, xjb)[cc % 2]
            hnd.wait()
            if cc + 1 < nch:
                hnd = pltpu.async_copy(
                    xg_hbm.at[sst.at[pl.ds((cc + 1) * CH, CH)]],
                    (xja, xjb)[(cc + 1) % 2], sems[(cc + 1) % 2])

            def group(g2, _):
                idx16 = cc * CH + g2 * 16 + it
                dl16 = plsc.load_gather(dstb, [idx16])
                a0 = plsc.load_gather(a0st, [idx16])
                a1 = plsc.load_gather(a1st, [idx16])
                p0 = _tanh16(a0 * P(0) + a1 * P(2) + P(4))
                p1 = _tanh16(a0 * P(1) + a1 * P(3) + P(5))
                ws = []
                for k in range(K):
                    d0 = p0 - P(6 + 2 * k)
                    d1 = p1 - P(7 + 2 * k)
                    gauss = (d0 * d0 * P(12 + 2 * k)
                             + d1 * d1 * P(13 + 2 * k))
                    ws.append(jnp.exp(-0.5 * gauss))
                w0, w1, w2 = ws
                for j in range(16):
                    w0b = _lane(w0, j)
                    w1b = _lane(w1, j)
                    w2b = _lane(w2, j)
                    rowb = _lane(dl16, j)
                    er16 = jnp.full((16,), g2 * 16 + j, jnp.int32)
                    msgs = []
                    for c in range(8):
                        col = c * 16 + it
                        x0 = plsc.load_gather(xj, [er16, col])
                        x1 = plsc.load_gather(xj, [er16, 128 + col])
                        x2 = plsc.load_gather(xj, [er16, 256 + col])
                        msgs.append(x0 * w0b + x1 * w1b + x2 * w2b)
                    accs = [plsc.load_gather(agg, [rowb, c * 16 + it])
                            for c in range(8)]
                    for c in range(8):
                        plsc.store_scatter(agg, [rowb, c * 16 + it],
                                           jnp.maximum(accs[c], msgs[c]))
                return 0

            lax.fori_loop(0, CH // 16, group, 0)
        return 0

    lax.fori_loop(0, cnt // SUP, super_body, 0)
    pltpu.sync_copy(agg.at[pl.ds(0, NPT)],
                    out_hbm.at[pl.ds(_al8(wid * NPT), NPT)])


@functools.partial(
    pl.kernel,
    out_type=jax.ShapeDtypeStruct((NPAD, D), jnp.float32),
    mesh=_MESH,
    compiler_params=_SC_PARAMS,
    scratch_types=[
        pltpu.VMEM((SUP,), jnp.int32),
        pltpu.VMEM((SUP,), jnp.int32),
        pltpu.VMEM((SUP,), jnp.float32),
        pltpu.VMEM((SUP,), jnp.float32),
        pltpu.VMEM((CH, K * D), jnp.float32),
        pltpu.VMEM((CH, K * D), jnp.float32),
        pltpu.VMEM((NPRM * 16,), jnp.float32),
        pltpu.VMEM((16,), jnp.int32),
        pltpu.VMEM((AGG_ROWS, D), jnp.float32),
        pltpu.SemaphoreType.DMA,
        pltpu.SemaphoreType.DMA,
    ],
)
def _sc_layer(*args):
    _layer_body(*args)


def _xg_body(h_ref, g_ref, out_ref):
    out_ref[...] = jnp.dot(h_ref[...], g_ref[...],
                           preferred_element_type=jnp.float32)


def _xg(h, g_l):
    return pl.pallas_call(
        _xg_body,
        out_shape=jax.ShapeDtypeStruct((N_NODES, K * D), jnp.float32),
    )(h, g_l)


def _xg_fused_body(a_ref, gb_ref, g_ref, out_ref):
    a = a_ref[...]
    hfix = jnp.where(a == -jnp.inf, 0.0, a) + gb_ref[...]
    h = jnp.maximum(hfix, 0.0)
    out_ref[...] = jnp.dot(h, g_ref[...], preferred_element_type=jnp.float32)


def _xg_fused(agg, gb_prev, g_l):
    return pl.pallas_call(
        _xg_fused_body,
        out_shape=jax.ShapeDtypeStruct((N_NODES, K * D), jnp.float32),
    )(agg, gb_prev.reshape(1, D), g_l)


def _head_body(a_ref, gb_ref, b_ref, fc1w_ref, fc1b_ref, fc2w_ref,
               fc2b_ref, out_ref):
    a = a_ref[...]
    h = jnp.maximum(jnp.where(a == -jnp.inf, 0.0, a) + gb_ref[...], 0.0)
    bcol = b_ref[...]
    gids = jax.lax.broadcasted_iota(jnp.int32, (N_NODES, NUM_GRAPHS), 1)
    onehot = (bcol == gids).astype(jnp.float32)
    sums = jnp.dot(onehot.T, h, preferred_element_type=jnp.float32)
    counts = jnp.sum(onehot, axis=0)
    hg = sums / jnp.clip(counts, 1.0)[:, None]
    hg = jnp.dot(hg, fc1w_ref[...], preferred_element_type=jnp.float32)
    hg = hg + fc1b_ref[...]
    hg = jnp.where(hg > 0, hg, jnp.exp(jnp.minimum(hg, 0.0)) - 1.0)
    hg = jnp.dot(hg, fc2w_ref[...], preferred_element_type=jnp.float32)
    hg = hg + fc2b_ref[...]
    m = jnp.max(hg, axis=0, keepdims=True)
    z = hg - m
    lse = jnp.log(jnp.sum(jnp.exp(z), axis=0, keepdims=True))
    out_ref[...] = z - lse


def _head(agg, gb_last, batch, fc1_w, fc1_b, fc2_w, fc2_b):
    return pl.pallas_call(
        _head_body,
        out_shape=jax.ShapeDtypeStruct((NUM_GRAPHS, OUT), jnp.float32),
    )(agg, gb_last.reshape(1, D), batch.reshape(N_NODES, 1),
      fc1_w, fc1_b.reshape(1, D), fc2_w, fc2_b.reshape(1, OUT))


def kernel(h, edge_attr, Wp, bp, g, mu, sigma, gb, fc1_w, fc1_b, fc2_w,
           fc2_b, edge_index, batch):
    src = edge_index[0]
    dst = edge_index[1]
    ea0 = edge_attr[:, 0]
    ea1 = edge_attr[:, 1]

    srcL, dstL, eaL0, eaL1, cnts2 = _compact(src, dst, ea0, ea1)

    agg = None
    for l in range(4):
        if l == 0:
            xg = _xg(h, g[0])
        else:
            xg = _xg_fused(agg, gb[l - 1], g[l])
        inv = 1.0 / (EPS + sigma[l] ** 2)
        pvec = jnp.concatenate([
            Wp[l].reshape(-1), bp[l].reshape(-1),
            mu[l].reshape(-1), inv.reshape(-1)]).astype(jnp.float32)
        prm = jnp.broadcast_to(pvec[:, None], (NPRM, 16)).reshape(-1)
        aggp = _sc_layer(xg, srcL, dstL, eaL0, eaL1, cnts2, prm)
        agg = aggp[:N_NODES]

    return _head(agg, gb[3], batch, fc1_w, fc1_b, fc2_w, fc2_b)


# speedup vs baseline: 4.2778x; 1.1464x over previous
import functools
import jax
import jax.numpy as jnp
from jax import lax
from jax.experimental import pallas as pl
from jax.experimental.pallas import tpu as pltpu
from jax.experimental.pallas import tpu_sc as plsc

N_NODES = 10000
N_EDGES = 320000
D = 128
K = 3
OUT = 10
NUM_GRAPHS = 64
EPS = 1e-15

NC = 2
NS = 16
NW = NC * NS
NPT = 320
NPAD = NW * NPT
SPARE = NPT
AGG_ROWS = NPT + 8
SCAN = 2000
FLUSH = 2048
BCAP = FLUSH + SCAN + 2048
SUP = 512
CH = 64
RCAP = N_EDGES + 4096 + 16
NPRM = 18

_MESH = plsc.VectorSubcoreMesh(core_axis_name="c", subcore_axis_name="s")
_SC_PARAMS = pltpu.CompilerParams(needs_layout_passes=False)


def _iota16():
    return lax.iota(jnp.int32, 16)


_GDN = lax.GatherDimensionNumbers(
    offset_dims=(), collapsed_slice_dims=(0,), start_index_map=(0,))


def _lane(v, j):
    idx = jnp.full((16, 1), j, jnp.int32)
    return lax.gather(v, idx, _GDN, (1,),
                      mode=lax.GatherScatterMode.PROMISE_IN_BOUNDS)


def _wid():
    return lax.axis_index("s") * NC + lax.axis_index("c")


def _al8(x):
    return pl.multiple_of(x, 8)


def _tanh16(x):
    e = jnp.exp(jnp.clip(x + x, -40.0, 40.0))
    return (e - 1.0) / (e + 1.0)


def _compact_body(src_hbm, dst_hbm, ea0_hbm, ea1_hbm,
                  srcL, dstL, eaL0, eaL1, cnts2,
                  sbuf2, dbuf2, abuf0, abuf1,
                  bsrc, bdl, bea0, bea1, cbuf, sem):
    wid = _wid()
    lo = wid * NPT
    it = _iota16()
    dummy_i = jnp.zeros((16,), jnp.int32)
    dummy_d = jnp.full((16,), SPARE, jnp.int32)
    dummy_f = jnp.zeros((16,), jnp.float32)

    bufs = (bsrc, bdl, bea0, bea1)
    outs = (srcL, dstL, eaL0, eaL1)

    def flush(carry):
        wptr, off = carry
        for b, o in zip(bufs, outs):
            pltpu.sync_copy(b.at[pl.ds(0, FLUSH)],
                            o.at[pl.ds(_al8(wid * RCAP + off), FLUSH)])
        rem = wptr - FLUSH
        mrem = it < rem
        for b in bufs:
            t = plsc.load_gather(b, [FLUSH + it])
            plsc.store_scatter(b, [it], t, mask=mrem)
        return rem, off + FLUSH

    def vec_step(v, carry):
        wptr, off = carry
        vals = []
        for h in range(2):
            b = v * 32 + h * 16 + it
            s16 = plsc.load_gather(sbuf2, [b])
            d16 = plsc.load_gather(dbuf2, [b])
            a0 = plsc.load_gather(abuf0, [b])
            a1 = plsc.load_gather(abuf1, [b])
            m = (d16 >= lo) & (d16 < lo + NPT)
            mi = m.astype(jnp.int32)
            cum = plsc.cumsum(mi)
            vals.append((s16, d16, a0, a1, m, mi, cum))
        base = wptr
        for s16, d16, a0, a1, m, mi, cum in vals:
            pos = base + cum - mi
            plsc.store_scatter(bsrc, [pos], s16, mask=m)
            plsc.store_scatter(bdl, [pos], d16 - lo, mask=m)
            plsc.store_scatter(bea0, [pos], a0, mask=m)
            plsc.store_scatter(bea1, [pos], a1, mask=m)
            base = base + _lane(cum, 15)
        return base, off

    def chunk(k, carry):
        pltpu.sync_copy(src_hbm.at[pl.ds(k * SCAN, SCAN)], sbuf2)
        pltpu.sync_copy(dst_hbm.at[pl.ds(k * SCAN, SCAN)], dbuf2)
        pltpu.sync_copy(ea0_hbm.at[pl.ds(k * SCAN, SCAN)], abuf0)
        pltpu.sync_copy(ea1_hbm.at[pl.ds(k * SCAN, SCAN)], abuf1)
        carry = lax.fori_loop(0, SCAN // 32, vec_step, carry)
        return lax.cond(jnp.max(carry[0]) >= FLUSH, flush, lambda c: c,
                        carry)

    wptr, off = lax.fori_loop(0, N_EDGES // SCAN, chunk,
                              (jnp.zeros((16,), jnp.int32), 0))

    pad1 = (-wptr) % 16
    mpad = it < pad1
    plsc.store_scatter(bsrc, [wptr + it], dummy_i, mask=mpad)
    plsc.store_scatter(bdl, [wptr + it], dummy_d, mask=mpad)
    plsc.store_scatter(bea0, [wptr + it], dummy_f, mask=mpad)
    plsc.store_scatter(bea1, [wptr + it], dummy_f, mask=mpad)
    wptr = wptr + pad1

    def padstep(_, w):
        def do(w):
            plsc.store_scatter(bsrc, [w + it], dummy_i)
            plsc.store_scatter(bdl, [w + it], dummy_d)
            plsc.store_scatter(bea0, [w + it], dummy_f)
            plsc.store_scatter(bea1, [w + it], dummy_f)
            return w + 16
        return lax.cond(jnp.max(w % SUP) != 0, do, lambda w: w, w)

    wptr = lax.fori_loop(0, SUP // 16 - 1, padstep, wptr)

    for b, o in zip(bufs, outs):
        pltpu.sync_copy(b.at[pl.ds(0, 4096)],
                        o.at[pl.ds(_al8(wid * RCAP + off), 4096)])

    cbuf[...] = (off + wptr).astype(jnp.int32)
    pltpu.sync_copy(cbuf, cnts2.at[pl.ds(_al8(wid * 16), 16)])


@functools.partial(
    pl.kernel,
    out_type=(
        jax.ShapeDtypeStruct((NW * RCAP,), jnp.int32),
        jax.ShapeDtypeStruct((NW * RCAP,), jnp.int32),
        jax.ShapeDtypeStruct((NW * RCAP,), jnp.float32),
        jax.ShapeDtypeStruct((NW * RCAP,), jnp.float32),
        jax.ShapeDtypeStruct((NW * 16,), jnp.int32),
    ),
    mesh=_MESH,
    compiler_params=_SC_PARAMS,
    scratch_types=[
        pltpu.VMEM((SCAN,), jnp.int32),
        pltpu.VMEM((SCAN,), jnp.int32),
        pltpu.VMEM((SCAN,), jnp.float32),
        pltpu.VMEM((SCAN,), jnp.float32),
        pltpu.VMEM((BCAP,), jnp.int32),
        pltpu.VMEM((BCAP,), jnp.int32),
        pltpu.VMEM((BCAP,), jnp.float32),
        pltpu.VMEM((BCAP,), jnp.float32),
        pltpu.VMEM((16,), jnp.int32),
        pltpu.SemaphoreType.DMA,
    ],
)
def _compact(*args):
    _compact_body(*args)


def _layer_body(xg_hbm, srcL, dstL, eaL0, eaL1, cnts2, prm_hbm, out_hbm,
                sst, dstb, a0st, a1st, xja, xjb, pv, wtmp, cbuf, agg,
                sema, semb):
    sems = (sema, semb)
    wid = _wid()
    it = _iota16()
    ninf = jnp.full((16,), -jnp.inf, jnp.float32)

    pltpu.sync_copy(cnts2.at[pl.ds(_al8(wid * 16), 16)], cbuf)
    cnt = jnp.max(cbuf[...])
    pltpu.sync_copy(prm_hbm, pv)

    def P(i):
        return pv[pl.ds(i * 16, 16)]

    def initr(r, _):
        rr = jnp.full((16,), r, jnp.int32)
        for c in range(8):
            plsc.store_scatter(agg, [rr, c * 16 + it], ninf)
        return 0

    lax.fori_loop(0, AGG_ROWS, initr, 0)

    def super_body(s, _):
        b0 = s * SUP
        pltpu.sync_copy(srcL.at[pl.ds(_al8(wid * RCAP + b0), SUP)], sst)
        pltpu.sync_copy(dstL.at[pl.ds(_al8(wid * RCAP + b0), SUP)], dstb)
        pltpu.sync_copy(eaL0.at[pl.ds(_al8(wid * RCAP + b0), SUP)], a0st)
        pltpu.sync_copy(eaL1.at[pl.ds(_al8(wid * RCAP + b0), SUP)], a1st)
        nch = SUP // CH
        hnd = pltpu.async_copy(
            xg_hbm.at[sst.at[pl.ds(0, CH)]], xja, sems[0])
        for cc in range(nch):
            xj = (xja, xjb)[cc % 2]
            hnd.wait()
            if cc + 1 < nch:
                hnd = pltpu.async_copy(
                    xg_hbm.at[sst.at[pl.ds((cc + 1) * CH, CH)]],
                    (xja, xjb)[(cc + 1) % 2], sems[(cc + 1) % 2])

            def group(g2, _):
                idx16 = cc * CH + g2 * 16 + it
                a0 = plsc.load_gather(a0st, [idx16])
                a1 = plsc.load_gather(a1st, [idx16])
                p0 = _tanh16(a0 * P(0) + a1 * P(2) + P(4))
                p1 = _tanh16(a0 * P(1) + a1 * P(3) + P(5))
                ws = []
                for k in range(K):
                    d0 = p0 - P(6 + 2 * k)
                    d1 = p1 - P(7 + 2 * k)
                    gauss = (d0 * d0 * P(12 + 2 * k)
                             + d1 * d1 * P(13 + 2 * k))
                    ws.append(jnp.exp(-0.5 * gauss))
                w0, w1, w2 = ws
                z16 = jnp.zeros((16,), jnp.int32)
                o16 = jnp.full((16,), 1, jnp.int32)
                t16 = jnp.full((16,), 2, jnp.int32)
                plsc.store_scatter(wtmp, [z16, it], w0)
                plsc.store_scatter(wtmp, [o16, it], w1)
                plsc.store_scatter(wtmp, [t16, it], w2)

                def edge(j, _):
                    jj = jnp.full((16,), cc * CH + g2 * 16 + j, jnp.int32)
                    jv = jnp.full((16,), j, jnp.int32)
                    rowb = plsc.load_gather(dstb, [jj])
                    w0b = plsc.load_gather(wtmp, [z16, jv])
                    w1b = plsc.load_gather(wtmp, [o16, jv])
                    w2b = plsc.load_gather(wtmp, [t16, jv])
                    er16 = jnp.full((16,), g2 * 16 + j, jnp.int32)
                    msgs = []
                    for c in range(8):
                        col = c * 16 + it
                        x0 = plsc.load_gather(xj, [er16, col])
                        x1 = plsc.load_gather(xj, [er16, 128 + col])
                        x2 = plsc.load_gather(xj, [er16, 256 + col])
                        msgs.append(x0 * w0b + x1 * w1b + x2 * w2b)
                    accs = [plsc.load_gather(agg, [rowb, c * 16 + it])
                            for c in range(8)]
                    for c in range(8):
                        plsc.store_scatter(agg, [rowb, c * 16 + it],
                                           jnp.maximum(accs[c], msgs[c]))
                    return 0

                lax.fori_loop(0, 16, edge, 0)
                return 0

            lax.fori_loop(0, CH // 16, group, 0)
        return 0

    lax.fori_loop(0, cnt // SUP, super_body, 0)
    pltpu.sync_copy(agg.at[pl.ds(0, NPT)],
                    out_hbm.at[pl.ds(_al8(wid * NPT), NPT)])


@functools.partial(
    pl.kernel,
    out_type=jax.ShapeDtypeStruct((NPAD, D), jnp.float32),
    mesh=_MESH,
    compiler_params=_SC_PARAMS,
    scratch_types=[
        pltpu.VMEM((SUP,), jnp.int32),
        pltpu.VMEM((SUP,), jnp.int32),
        pltpu.VMEM((SUP,), jnp.float32),
        pltpu.VMEM((SUP,), jnp.float32),
        pltpu.VMEM((CH, K * D), jnp.float32),
        pltpu.VMEM((CH, K * D), jnp.float32),
        pltpu.VMEM((NPRM * 16,), jnp.float32),
        pltpu.VMEM((K, 16), jnp.float32),
        pltpu.VMEM((16,), jnp.int32),
        pltpu.VMEM((AGG_ROWS, D), jnp.float32),
        pltpu.SemaphoreType.DMA,
        pltpu.SemaphoreType.DMA,
    ],
)
def _sc_layer(*args):
    _layer_body(*args)


def _xg_body(h_ref, g_ref, out_ref):
    out_ref[...] = jnp.dot(h_ref[...], g_ref[...],
                           preferred_element_type=jnp.float32)


def _xg(h, g_l):
    return pl.pallas_call(
        _xg_body,
        out_shape=jax.ShapeDtypeStruct((N_NODES, K * D), jnp.float32),
    )(h, g_l)


def _xg_fused_body(a_ref, gb_ref, g_ref, out_ref):
    a = a_ref[...]
    hfix = jnp.where(a == -jnp.inf, 0.0, a) + gb_ref[...]
    h = jnp.maximum(hfix, 0.0)
    out_ref[...] = jnp.dot(h, g_ref[...], preferred_element_type=jnp.float32)


def _xg_fused(agg, gb_prev, g_l):
    return pl.pallas_call(
        _xg_fused_body,
        out_shape=jax.ShapeDtypeStruct((N_NODES, K * D), jnp.float32),
    )(agg, gb_prev.reshape(1, D), g_l)


def _head_body(a_ref, gb_ref, b_ref, fc1w_ref, fc1b_ref, fc2w_ref,
               fc2b_ref, out_ref):
    a = a_ref[...]
    h = jnp.maximum(jnp.where(a == -jnp.inf, 0.0, a) + gb_ref[...], 0.0)
    bcol = b_ref[...]
    gids = jax.lax.broadcasted_iota(jnp.int32, (N_NODES, NUM_GRAPHS), 1)
    onehot = (bcol == gids).astype(jnp.float32)
    sums = jnp.dot(onehot.T, h, preferred_element_type=jnp.float32)
    counts = jnp.sum(onehot, axis=0)
    hg = sums / jnp.clip(counts, 1.0)[:, None]
    hg = jnp.dot(hg, fc1w_ref[...], preferred_element_type=jnp.float32)
    hg = hg + fc1b_ref[...]
    hg = jnp.where(hg > 0, hg, jnp.exp(jnp.minimum(hg, 0.0)) - 1.0)
    hg = jnp.dot(hg, fc2w_ref[...], preferred_element_type=jnp.float32)
    hg = hg + fc2b_ref[...]
    m = jnp.max(hg, axis=0, keepdims=True)
    z = hg - m
    lse = jnp.log(jnp.sum(jnp.exp(z), axis=0, keepdims=True))
    out_ref[...] = z - lse


def _head(agg, gb_last, batch, fc1_w, fc1_b, fc2_w, fc2_b):
    return pl.pallas_call(
        _head_body,
        out_shape=jax.ShapeDtypeStruct((NUM_GRAPHS, OUT), jnp.float32),
    )(agg, gb_last.reshape(1, D), batch.reshape(N_NODES, 1),
      fc1_w, fc1_b.reshape(1, D), fc2_w, fc2_b.reshape(1, OUT))


def kernel(h, edge_attr, Wp, bp, g, mu, sigma, gb, fc1_w, fc1_b, fc2_w,
           fc2_b, edge_index, batch):
    src = edge_index[0]
    dst = edge_index[1]
    ea0 = edge_attr[:, 0]
    ea1 = edge_attr[:, 1]

    srcL, dstL, eaL0, eaL1, cnts2 = _compact(src, dst, ea0, ea1)

    agg = None
    for l in range(4):
        if l == 0:
            xg = _xg(h, g[0])
        else:
            xg = _xg_fused(agg, gb[l - 1], g[l])
        inv = 1.0 / (EPS + sigma[l] ** 2)
        pvec = jnp.concatenate([
            Wp[l].reshape(-1), bp[l].reshape(-1),
            mu[l].reshape(-1), inv.reshape(-1)]).astype(jnp.float32)
        prm = jnp.broadcast_to(pvec[:, None], (NPRM, 16)).reshape(-1)
        aggp = _sc_layer(xg, srcL, dstL, eaL0, eaL1, cnts2, prm)
        agg = aggp[:N_NODES]

    return _head(agg, gb[3], batch, fc1_w, fc1_b, fc2_w, fc2_b)
